# baseline (speedup 1.0000x reference)
import numpy as np
import jax
import jax.numpy as jnp
from jax.experimental import pallas as pl
from jax.experimental.pallas import tpu as pltpu

Q = 784
D = 1536
K = 65536
KB_A = 2048
CHUNK_A = 256
KB_B = 4096
FMAP = 28
IMG = 224
_BIG = 3.0e38
_INV_NORM = float(1.0 / np.sqrt(float(D)))
_NT = (((1,), (1,)), ((), ()))


def _resize_weight_mat() -> np.ndarray:
    inv_scale = FMAP / IMG
    sample_f = (np.arange(IMG) + 0.5) * inv_scale - 0.5
    x = np.abs(sample_f[None, :] - np.arange(FMAP)[:, None])
    weights = np.maximum(0.0, 1.0 - x)
    total = weights.sum(axis=0, keepdims=True)
    weights = np.where(np.abs(total) > 1e-6, weights / total, 0.0)
    ok = (sample_f >= -1.0) & (sample_f <= FMAP)
    weights = np.where(ok[None, :], weights, 0.0)
    return np.ascontiguousarray(weights.T.astype(np.float32))


_R_MAT = _resize_weight_mat()


def _phase_a_kernel(amh_ref, aml_ref, patch_ref, bank_ref, dist_ref,
                    sidx_ref, istar_ref, sstar_ref, minv_ref, mini_ref):
    j = pl.program_id(0)
    amh = amh_ref[...]
    aml = aml_ref[...]

    @pl.when(j == 0)
    def _init():
        minv_ref[...] = jnp.full((1, Q), _BIG, jnp.float32)
        mini_ref[...] = jnp.zeros((1, Q), jnp.int32)

    ones8 = jnp.ones((8, D), jnp.bfloat16)

    def _chunk(c, carry):
        b = bank_ref[pl.ds(c * CHUNK_A, CHUNK_A), :]
        bh = b.astype(jnp.bfloat16)
        bl = (b - bh.astype(jnp.float32)).astype(jnp.bfloat16)
        bsq = b * b
        bsqh = bsq.astype(jnp.bfloat16)
        bsql = (bsq - bsqh.astype(jnp.float32)).astype(jnp.bfloat16)
        b2c = jax.lax.dot_general(bsqh, ones8, _NT,
                                  preferred_element_type=jnp.float32)
        b2c += jax.lax.dot_general(bsql, ones8, _NT,
                                   preferred_element_type=jnp.float32)
        p = jax.lax.dot_general(bh, amh, _NT,
                                preferred_element_type=jnp.float32)
        p += jax.lax.dot_general(bl, amh, _NT,
                                 preferred_element_type=jnp.float32)
        p += jax.lax.dot_general(bh, aml, _NT,
                                 preferred_element_type=jnp.float32)
        scores = p + b2c[:, 0:1]
        blk_min = jnp.min(scores, axis=0, keepdims=True)
        blk_arg = (jnp.argmin(scores, axis=0, keepdims=True)
                   .astype(jnp.int32) + (j * KB_A + c * CHUNK_A))
        prev = minv_ref[...]
        take = blk_min < prev
        minv_ref[...] = jnp.where(take, blk_min, prev)
        mini_ref[...] = jnp.where(take, blk_arg, mini_ref[...])
        return carry

    jax.lax.fori_loop(0, KB_A // CHUNK_A, _chunk, 0, unroll=False)

    @pl.when(j == pl.num_programs(0) - 1)
    def _finish():
        a = patch_ref[...]
        asq = a * a
        asqh = asq.astype(jnp.bfloat16)
        asql = (asq - asqh.astype(jnp.float32)).astype(jnp.bfloat16)
        a2r = jax.lax.dot_general(ones8, asqh, _NT,
                                  preferred_element_type=jnp.float32)
        a2r += jax.lax.dot_general(ones8, asql, _NT,
                                   preferred_element_type=jnp.float32)
        a2 = a2r[0:1, :]
        d2 = minv_ref[...] + a2
        dist = jnp.sqrt(jnp.maximum(d2, 1e-12))
        dist_ref[...] = dist
        smax = jnp.max(dist, keepdims=True)
        lit = jax.lax.broadcasted_iota(jnp.int32, (1, Q), 1)
        sidx = jnp.min(jnp.where(dist == smax, lit, Q), keepdims=True)
        istar = jnp.sum(jnp.where(lit == sidx, mini_ref[...], 0),
                        keepdims=True)
        sstar_ref[...] = smax
        sidx_ref[...] = sidx
        istar_ref[...] = istar


def _phase_b_kernel(mvm_ref, bank_ref, d2_ref):
    b = bank_ref[...]
    bh = b.astype(jnp.bfloat16)
    bsqh = (b * b).astype(jnp.bfloat16)
    ones8 = jnp.ones((8, D), jnp.bfloat16)
    b2r = jax.lax.dot_general(ones8, bsqh, _NT,
                              preferred_element_type=jnp.float32)
    prod = jax.lax.dot_general(mvm_ref[...], bh, _NT,
                               preferred_element_type=jnp.float32)
    d2_ref[...] = prod + b2r[0:1, :]


def _phase_c_kernel(d2_ref, sstar_ref, m2t_ref, m28_ref, r_ref,
                    s_ref, segm_ref):
    dstar = d2_ref[0:1, :]
    dtest = d2_ref[1:2, :]
    it = jax.lax.broadcasted_iota(jnp.int32, (1, K), 1)

    cur = dstar
    tvs = []
    for _ in range(3):
        pos = jnp.argmin(cur, axis=1, keepdims=True).astype(jnp.int32)
        tvs.append(jnp.sum(jnp.where(it == pos, dtest, 0.0), keepdims=True))
        cur = jnp.where(it == pos, _BIG, cur)

    m2t = m2t_ref[...]
    wd1 = jnp.sqrt(tvs[1] + m2t)
    wd2 = jnp.sqrt(tvs[2] + m2t)
    ss = sstar_ref[...]
    w = 1.0 - jnp.exp(ss * _INV_NORM) / (jnp.exp(wd1 * _INV_NORM) +
                                         jnp.exp(wd2 * _INV_NORM))
    s_ref[...] = w * ss

    r = r_ref[...]
    m28 = m28_ref[...]
    tmp = jax.lax.dot_general(
        m28, r, (((1,), (1,)), ((), ())),
        preferred_element_type=jnp.float32)
    segm_ref[...] = jax.lax.dot_general(
        r, tmp, (((1,), (0,)), ((), ())),
        preferred_element_type=jnp.float32)


def kernel(patch, memory_bank):
    am = -2.0 * patch
    am_hi = am.astype(jnp.bfloat16)
    am_lo = (am - am_hi.astype(jnp.float32)).astype(jnp.bfloat16)

    nb_a = K // KB_A
    dist, sidx, istar, sstar = pl.pallas_call(
        _phase_a_kernel,
        grid=(nb_a,),
        in_specs=[
            pl.BlockSpec((Q, D), lambda j: (0, 0)),
            pl.BlockSpec((Q, D), lambda j: (0, 0)),
            pl.BlockSpec((Q, D), lambda j: (0, 0)),
            pl.BlockSpec((KB_A, D), lambda j: (j, 0)),
        ],
        out_specs=[
            pl.BlockSpec((1, Q), lambda j: (0, 0)),
            pl.BlockSpec((1, 1), lambda j: (0, 0)),
            pl.BlockSpec((1, 1), lambda j: (0, 0)),
            pl.BlockSpec((1, 1), lambda j: (0, 0)),
        ],
        out_shape=[
            jax.ShapeDtypeStruct((1, Q), jnp.float32),
            jax.ShapeDtypeStruct((1, 1), jnp.int32),
            jax.ShapeDtypeStruct((1, 1), jnp.int32),
            jax.ShapeDtypeStruct((1, 1), jnp.float32),
        ],
        scratch_shapes=[
            pltpu.VMEM((1, Q), jnp.float32),
            pltpu.VMEM((1, Q), jnp.int32),
        ],
        compiler_params=pltpu.CompilerParams(
            dimension_semantics=("arbitrary",)),
    )(am_hi, am_lo, patch, memory_bank)

    m_star = jax.lax.dynamic_slice(memory_bank, (istar[0, 0], 0), (1, D))
    m_test = jax.lax.dynamic_slice(patch, (sidx[0, 0], 0), (1, D))
    mvm = (-2.0 * jnp.concatenate([m_star, m_test], axis=0)
           ).astype(jnp.bfloat16)
    m2t = jnp.sum(m_test * m_test, axis=1, keepdims=True)

    nb_b = K // KB_B
    d2 = pl.pallas_call(
        _phase_b_kernel,
        grid=(nb_b,),
        in_specs=[
            pl.BlockSpec((2, D), lambda j: (0, 0)),
            pl.BlockSpec((KB_B, D), lambda j: (j, 0)),
        ],
        out_specs=pl.BlockSpec((2, KB_B), lambda j: (0, j)),
        out_shape=jax.ShapeDtypeStruct((2, K), jnp.float32),
        compiler_params=pltpu.CompilerParams(
            dimension_semantics=("arbitrary",)),
    )(mvm, memory_bank)

    m28 = dist.reshape(FMAP, FMAP)
    r_mat = jnp.asarray(_R_MAT)
    s, segm = pl.pallas_call(
        _phase_c_kernel,
        out_shape=[
            jax.ShapeDtypeStruct((1, 1), jnp.float32),
            jax.ShapeDtypeStruct((IMG, IMG), jnp.float32),
        ],
    )(d2, sstar, m2t, m28, r_mat)

    return (s[0, 0], segm.reshape(1, 1, IMG, IMG))


# speedup vs baseline: 1.2206x; 1.2206x over previous
import numpy as np
import jax
import jax.numpy as jnp
from jax.experimental import pallas as pl
from jax.experimental.pallas import tpu as pltpu

Q = 784
D = 1536
K = 65536
KB_A = 2048
CHUNK_A = 1024
KB_B = 4096
FMAP = 28
IMG = 224
_BIG = 3.0e38
_INV_NORM = float(1.0 / np.sqrt(float(D)))
_NT = (((1,), (1,)), ((), ()))


def _resize_weight_mat() -> np.ndarray:
    inv_scale = FMAP / IMG
    sample_f = (np.arange(IMG) + 0.5) * inv_scale - 0.5
    x = np.abs(sample_f[None, :] - np.arange(FMAP)[:, None])
    weights = np.maximum(0.0, 1.0 - x)
    total = weights.sum(axis=0, keepdims=True)
    weights = np.where(np.abs(total) > 1e-6, weights / total, 0.0)
    ok = (sample_f >= -1.0) & (sample_f <= FMAP)
    weights = np.where(ok[None, :], weights, 0.0)
    return np.ascontiguousarray(weights.T.astype(np.float32))


_R_MAT = _resize_weight_mat()


def _phase_a_kernel(ams_ref, patch_ref, bank_ref, dist_ref,
                    sidx_ref, istar_ref, sstar_ref, minv_ref, mini_ref):
    j = pl.program_id(0)
    ams = ams_ref[...]
    amh = ams[0:Q, :]

    @pl.when(j == 0)
    def _init():
        minv_ref[...] = jnp.full((1, Q), _BIG, jnp.float32)
        mini_ref[...] = jnp.zeros((1, Q), jnp.int32)

    ones8 = jnp.ones((8, D), jnp.bfloat16)

    def _chunk(c, carry):
        b = bank_ref[pl.ds(c * CHUNK_A, CHUNK_A), :]
        bh = b.astype(jnp.bfloat16)
        bl = (b - bh.astype(jnp.float32)).astype(jnp.bfloat16)
        b2c = jnp.sum(b * b, axis=1, keepdims=True)
        p2 = jax.lax.dot_general(bh, ams, _NT,
                                 preferred_element_type=jnp.float32)
        p = p2[:, 0:Q] + p2[:, Q:2 * Q]
        p += jax.lax.dot_general(bl, amh, _NT,
                                 preferred_element_type=jnp.float32)
        scores = p + b2c
        blk_min = jnp.min(scores, axis=0, keepdims=True)
        blk_arg = (jnp.argmin(scores, axis=0, keepdims=True)
                   .astype(jnp.int32) + (j * KB_A + c * CHUNK_A))
        prev = minv_ref[...]
        take = blk_min < prev
        minv_ref[...] = jnp.where(take, blk_min, prev)
        mini_ref[...] = jnp.where(take, blk_arg, mini_ref[...])
        return carry

    jax.lax.fori_loop(0, KB_A // CHUNK_A, _chunk, 0, unroll=False)

    @pl.when(j == pl.num_programs(0) - 1)
    def _finish():
        a = patch_ref[...]
        asq = a * a
        asqh = asq.astype(jnp.bfloat16)
        asql = (asq - asqh.astype(jnp.float32)).astype(jnp.bfloat16)
        a2r = jax.lax.dot_general(ones8, asqh, _NT,
                                  preferred_element_type=jnp.float32)
        a2r += jax.lax.dot_general(ones8, asql, _NT,
                                   preferred_element_type=jnp.float32)
        a2 = a2r[0:1, :]
        d2 = minv_ref[...] + a2
        dist = jnp.sqrt(jnp.maximum(d2, 1e-12))
        dist_ref[...] = dist
        smax = jnp.max(dist, keepdims=True)
        lit = jax.lax.broadcasted_iota(jnp.int32, (1, Q), 1)
        sidx = jnp.min(jnp.where(dist == smax, lit, Q), keepdims=True)
        istar = jnp.sum(jnp.where(lit == sidx, mini_ref[...], 0),
                        keepdims=True)
        sstar_ref[...] = smax
        sidx_ref[...] = sidx
        istar_ref[...] = istar


def _phase_b_kernel(mvm_ref, bank_ref, d2_ref):
    b = bank_ref[...]
    bh = b.astype(jnp.bfloat16)
    bsqh = (b * b).astype(jnp.bfloat16)
    ones8 = jnp.ones((8, D), jnp.bfloat16)
    b2r = jax.lax.dot_general(ones8, bsqh, _NT,
                              preferred_element_type=jnp.float32)
    prod = jax.lax.dot_general(mvm_ref[...], bh, _NT,
                               preferred_element_type=jnp.float32)
    d2_ref[...] = prod + b2r[0:1, :]


def _phase_c_kernel(d2_ref, sstar_ref, m2t_ref, m28_ref, r_ref,
                    s_ref, segm_ref):
    dstar = d2_ref[0:1, :]
    dtest = d2_ref[1:2, :]
    it = jax.lax.broadcasted_iota(jnp.int32, (1, K), 1)

    cur = dstar
    tvs = []
    for _ in range(3):
        pos = jnp.argmin(cur, axis=1, keepdims=True).astype(jnp.int32)
        tvs.append(jnp.sum(jnp.where(it == pos, dtest, 0.0), keepdims=True))
        cur = jnp.where(it == pos, _BIG, cur)

    m2t = m2t_ref[...]
    wd1 = jnp.sqrt(tvs[1] + m2t)
    wd2 = jnp.sqrt(tvs[2] + m2t)
    ss = sstar_ref[...]
    w = 1.0 - jnp.exp(ss * _INV_NORM) / (jnp.exp(wd1 * _INV_NORM) +
                                         jnp.exp(wd2 * _INV_NORM))
    s_ref[...] = w * ss

    r = r_ref[...]
    m28 = m28_ref[...]
    tmp = jax.lax.dot_general(
        m28, r, (((1,), (1,)), ((), ())),
        preferred_element_type=jnp.float32)
    segm_ref[...] = jax.lax.dot_general(
        r, tmp, (((1,), (0,)), ((), ())),
        preferred_element_type=jnp.float32)


def kernel(patch, memory_bank):
    am = -2.0 * patch
    am_hi = am.astype(jnp.bfloat16)
    am_lo = (am - am_hi.astype(jnp.float32)).astype(jnp.bfloat16)
    am_stack = jnp.concatenate([am_hi, am_lo], axis=0)

    nb_a = K // KB_A
    dist, sidx, istar, sstar = pl.pallas_call(
        _phase_a_kernel,
        grid=(nb_a,),
        in_specs=[
            pl.BlockSpec((2 * Q, D), lambda j: (0, 0)),
            pl.BlockSpec((Q, D), lambda j: (0, 0)),
            pl.BlockSpec((KB_A, D), lambda j: (j, 0)),
        ],
        out_specs=[
            pl.BlockSpec((1, Q), lambda j: (0, 0)),
            pl.BlockSpec((1, 1), lambda j: (0, 0)),
            pl.BlockSpec((1, 1), lambda j: (0, 0)),
            pl.BlockSpec((1, 1), lambda j: (0, 0)),
        ],
        out_shape=[
            jax.ShapeDtypeStruct((1, Q), jnp.float32),
            jax.ShapeDtypeStruct((1, 1), jnp.int32),
            jax.ShapeDtypeStruct((1, 1), jnp.int32),
            jax.ShapeDtypeStruct((1, 1), jnp.float32),
        ],
        scratch_shapes=[
            pltpu.VMEM((1, Q), jnp.float32),
            pltpu.VMEM((1, Q), jnp.int32),
        ],
        compiler_params=pltpu.CompilerParams(
            dimension_semantics=("arbitrary",)),
    )(am_stack, patch, memory_bank)

    m_star = jax.lax.dynamic_slice(memory_bank, (istar[0, 0], 0), (1, D))
    m_test = jax.lax.dynamic_slice(patch, (sidx[0, 0], 0), (1, D))
    mvm = (-2.0 * jnp.concatenate([m_star, m_test], axis=0)
           ).astype(jnp.bfloat16)
    m2t = jnp.sum(m_test * m_test, axis=1, keepdims=True)

    nb_b = K // KB_B
    d2 = pl.pallas_call(
        _phase_b_kernel,
        grid=(nb_b,),
        in_specs=[
            pl.BlockSpec((2, D), lambda j: (0, 0)),
            pl.BlockSpec((KB_B, D), lambda j: (j, 0)),
        ],
        out_specs=pl.BlockSpec((2, KB_B), lambda j: (0, j)),
        out_shape=jax.ShapeDtypeStruct((2, K), jnp.float32),
        compiler_params=pltpu.CompilerParams(
            dimension_semantics=("arbitrary",)),
    )(mvm, memory_bank)

    m28 = dist.reshape(FMAP, FMAP)
    r_mat = jnp.asarray(_R_MAT)
    s, segm = pl.pallas_call(
        _phase_c_kernel,
        out_shape=[
            jax.ShapeDtypeStruct((1, 1), jnp.float32),
            jax.ShapeDtypeStruct((IMG, IMG), jnp.float32),
        ],
    )(d2, sstar, m2t, m28, r_mat)

    return (s[0, 0], segm.reshape(1, 1, IMG, IMG))
